# baseline (speedup 1.0000x reference)
import functools

import jax
import jax.numpy as jnp
from jax import lax
from jax.experimental import pallas as pl
from jax.experimental.pallas import tpu as pltpu

N = 10000
E = 160000
D = 256
DH = 128
EDGE_CLASSES = 5

RN = 1000
RE = 2000


def _dot(a, b):
    return lax.dot_general(a, b, (((1,), (1,)), ((), ())),
                           preferred_element_type=jnp.float32)


def _input_proj_body(h_ref, w0_ref, w1_ref, c_ref, g_ref, b_ref, o_ref):
    eps = 1e-5
    for ci, w_ref in ((0, w0_ref), (1, w1_ref)):
        x = h_ref[:, ci * DH:(ci + 1) * DH]
        z = _dot(x, w_ref[...]) + c_ref[0, ci * DH:(ci + 1) * DH]
        mu = jnp.mean(z, axis=1, keepdims=True)
        zc = z - mu
        var = jnp.mean(zc * zc, axis=1, keepdims=True)
        zn = zc * lax.rsqrt(var + eps)
        zn = zn * g_ref[0, ci * DH:(ci + 1) * DH] + b_ref[0, ci * DH:(ci + 1) * DH]
        o_ref[ci] = jnp.maximum(zn, 0.0)


def _input_proj(h, w0, w1, c, g, b):
    grid = (N // RN,)
    return pl.pallas_call(
        _input_proj_body,
        grid=grid,
        in_specs=[
            pl.BlockSpec((RN, D), lambda i: (i, 0)),
            pl.BlockSpec((DH, DH), lambda i: (0, 0)),
            pl.BlockSpec((DH, DH), lambda i: (0, 0)),
            pl.BlockSpec((1, D), lambda i: (0, 0)),
            pl.BlockSpec((1, D), lambda i: (0, 0)),
            pl.BlockSpec((1, D), lambda i: (0, 0)),
        ],
        out_specs=pl.BlockSpec((2, RN, DH), lambda i: (0, i, 0)),
        out_shape=jax.ShapeDtypeStruct((2, N, DH), jnp.float32),
    )(h, w0, w1, c, g, b)


def _sage_body(hh_ref, agg_ref, deg_ref, ws_ref, bs_ref, wn_ref, g_ref,
               be_ref, o_ref):
    eps = 1e-5
    deg = deg_ref[0, :, 0:1] + deg_ref[1, :, 0:1]
    inv = jnp.where(deg > 0, 1.0 / jnp.maximum(deg, 1.0), 0.0)
    x0 = hh_ref[0]
    x1 = hh_ref[1]
    m0 = agg_ref[0] * inv
    m1 = agg_ref[1] * inv
    halves = []
    for j in (0, 1):
        r = (_dot(x0, ws_ref[j * DH:(j + 1) * DH, 0:DH])
             + _dot(x1, ws_ref[j * DH:(j + 1) * DH, DH:D])
             + _dot(m0, wn_ref[j * DH:(j + 1) * DH, 0:DH])
             + _dot(m1, wn_ref[j * DH:(j + 1) * DH, DH:D])
             + bs_ref[0, j * DH:(j + 1) * DH])
        halves.append(jnp.maximum(r, 0.0))
    r0, r1 = halves
    mu = (jnp.sum(r0, axis=1, keepdims=True)
          + jnp.sum(r1, axis=1, keepdims=True)) / D
    c0 = r0 - mu
    c1 = r1 - mu
    var = (jnp.sum(c0 * c0, axis=1, keepdims=True)
           + jnp.sum(c1 * c1, axis=1, keepdims=True)) / D
    rs = lax.rsqrt(var + eps)
    o_ref[0] = c0 * rs * g_ref[0, 0:DH] + be_ref[0, 0:DH]
    o_ref[1] = c1 * rs * g_ref[0, DH:D] + be_ref[0, DH:D]


def _sage_layer(hh, agg, degp, ws, bs, wn, g, be):
    grid = (N // RN,)
    return pl.pallas_call(
        _sage_body,
        grid=grid,
        in_specs=[
            pl.BlockSpec((2, RN, DH), lambda i: (0, i, 0)),
            pl.BlockSpec((2, RN, DH), lambda i: (0, i, 0)),
            pl.BlockSpec((2, RN, 16), lambda i: (0, i, 0)),
            pl.BlockSpec((D, D), lambda i: (0, 0)),
            pl.BlockSpec((1, D), lambda i: (0, 0)),
            pl.BlockSpec((D, D), lambda i: (0, 0)),
            pl.BlockSpec((1, D), lambda i: (0, 0)),
            pl.BlockSpec((1, D), lambda i: (0, 0)),
        ],
        out_specs=pl.BlockSpec((2, RN, DH), lambda i: (0, i, 0)),
        out_shape=jax.ShapeDtypeStruct((2, N, DH), jnp.float32),
    )(hh, agg, degp, ws, bs, wn, g, be)


def _pred_tables_body(hh_ref, w1_ref, o_ref):
    x0 = hh_ref[0]
    x1 = hh_ref[1]
    for t in range(4):
        off = (t // 2) * D
        j = t % 2
        o_ref[t] = (_dot(x0, w1_ref[j * DH:(j + 1) * DH, off:off + DH])
                    + _dot(x1, w1_ref[j * DH:(j + 1) * DH, off + DH:off + D]))


def _pred_tables(hh, w1):
    grid = (N // RN,)
    return pl.pallas_call(
        _pred_tables_body,
        grid=grid,
        in_specs=[
            pl.BlockSpec((2, RN, DH), lambda i: (0, i, 0)),
            pl.BlockSpec((D, 2 * D), lambda i: (0, 0)),
        ],
        out_specs=pl.BlockSpec((4, RN, DH), lambda i: (0, i, 0)),
        out_shape=jax.ShapeDtypeStruct((4, N, DH), jnp.float32),
    )(hh, w1)


def _edge_head_body(x_ref, ef_ref, b1_ref, g_ref, be_ref, w2_ref, b2_ref,
                    o_ref):
    eps = 1e-5
    x0 = x_ref[0] + b1_ref[0, 0:DH]
    x1 = x_ref[1] + b1_ref[0, DH:D]
    mu = (jnp.sum(x0, axis=1, keepdims=True)
          + jnp.sum(x1, axis=1, keepdims=True)) / D
    c0 = x0 - mu
    c1 = x1 - mu
    var = (jnp.sum(c0 * c0, axis=1, keepdims=True)
           + jnp.sum(c1 * c1, axis=1, keepdims=True)) / D
    rs = lax.rsqrt(var + eps)
    n0 = jnp.maximum(c0 * rs * g_ref[0, 0:DH] + be_ref[0, 0:DH], 0.0)
    n1 = jnp.maximum(c1 * rs * g_ref[0, DH:D] + be_ref[0, DH:D], 0.0)
    o_ref[...] = (_dot(n0, w2_ref[:, 0:DH])
                  + _dot(n1, w2_ref[:, DH:D])
                  + _dot(ef_ref[...], w2_ref[:, D:D + 2])
                  + b2_ref[0, :])


def _edge_head(x, ef, b1, g, be, w2, b2):
    grid = (E // RE,)
    return pl.pallas_call(
        _edge_head_body,
        grid=grid,
        in_specs=[
            pl.BlockSpec((2, RE, DH), lambda i: (0, i, 0)),
            pl.BlockSpec((RE, 2), lambda i: (i, 0)),
            pl.BlockSpec((1, D), lambda i: (0, 0)),
            pl.BlockSpec((1, D), lambda i: (0, 0)),
            pl.BlockSpec((1, D), lambda i: (0, 0)),
            pl.BlockSpec((8, D + 2), lambda i: (0, 0)),
            pl.BlockSpec((1, 8), lambda i: (0, 0)),
        ],
        out_specs=pl.BlockSpec((RE, 8), lambda i: (i, 0)),
        out_shape=jax.ShapeDtypeStruct((E, 8), jnp.float32),
    )(x, ef, b1, g, be, w2, b2)


def kernel(h, edge_weight, edge_feat, params, edge_index):
    p = params
    src = edge_index[0]
    dst = edge_index[1]

    c = jnp.concatenate([p['cp0'], p['cp1']])[None, :]
    g_in = jnp.concatenate([p['gp0'], p['gp1']])[None, :]
    b_in = jnp.concatenate([p['betap0'], p['betap1']])[None, :]
    hh = _input_proj(h, p['Wp0'], p['Wp1'], c, g_in, b_in)

    deg = jax.ops.segment_sum(jnp.ones((E,), jnp.float32), dst, num_segments=N)
    degp = jnp.stack([jnp.broadcast_to(deg[:, None], (N, 16)),
                      jnp.zeros((N, 16), jnp.float32)])

    for l in range(3):
        hhc = jnp.concatenate([hh[0], hh[1]], axis=1)
        msg = hhc[src] * edge_weight[:, None]
        agg = jax.ops.segment_sum(msg, dst, num_segments=N)
        agg2 = jnp.stack([agg[:, :DH], agg[:, DH:]])
        hh = _sage_layer(hh, agg2, degp, p[f'Wself{l}'], p[f'bself{l}'][None, :],
                         p[f'Wneigh{l}'], p[f'g{l}'][None, :],
                         p[f'beta{l}'][None, :])

    tab = _pred_tables(hh, p['W1'])

    a = jnp.concatenate([tab[0], tab[1]], axis=1)
    b = jnp.concatenate([tab[2], tab[3]], axis=1)
    xx = a[src] + b[dst]
    x2 = jnp.stack([xx[:, :DH], xx[:, DH:]])

    w2p = jnp.zeros((8, D + 2), jnp.float32).at[:EDGE_CLASSES].set(p['W2'])
    b2p = jnp.zeros((1, 8), jnp.float32).at[0, :EDGE_CLASSES].set(p['b2'])
    out8 = _edge_head(x2, edge_feat, p['b1'][None, :], p['g_pred'][None, :],
                      p['beta_pred'][None, :], w2p, b2p)
    return out8[:, :EDGE_CLASSES]


# speedup vs baseline: 2.5198x; 2.5198x over previous
import functools

import jax
import jax.numpy as jnp
from jax import lax
from jax.experimental import pallas as pl
from jax.experimental.pallas import tpu as pltpu
from jax.experimental.pallas import tpu_sc as plsc

N = 10000
E = 160000
D = 256
DH = 128
EDGE_CLASSES = 5

RN = 1000
RE = 2000

NSUB = 16
NP = 10240
NPS = NP // NSUB
CA = 80
EPS_A = E // NSUB
TA = EPS_A // CA
CD = 40
EPS_D = E // 32
TD = EPS_D // CD


def _dot(a, b):
    return lax.dot_general(a, b, (((1,), (1,)), ((), ())),
                           preferred_element_type=jnp.float32)


def _input_proj_body(h_ref, w0_ref, w1_ref, c_ref, g_ref, b_ref, o_ref):
    eps = 1e-5
    for ci, w_ref in ((0, w0_ref), (1, w1_ref)):
        x = h_ref[:, ci * DH:(ci + 1) * DH]
        z = _dot(x, w_ref[...]) + c_ref[0, ci * DH:(ci + 1) * DH]
        mu = jnp.mean(z, axis=1, keepdims=True)
        zc = z - mu
        var = jnp.mean(zc * zc, axis=1, keepdims=True)
        zn = zc * lax.rsqrt(var + eps)
        zn = zn * g_ref[0, ci * DH:(ci + 1) * DH] + b_ref[0, ci * DH:(ci + 1) * DH]
        o_ref[ci] = jnp.maximum(zn, 0.0)


def _input_proj(h, w0, w1, c, g, b):
    grid = (N // RN,)
    return pl.pallas_call(
        _input_proj_body,
        grid=grid,
        in_specs=[
            pl.BlockSpec((RN, D), lambda i: (i, 0)),
            pl.BlockSpec((DH, DH), lambda i: (0, 0)),
            pl.BlockSpec((DH, DH), lambda i: (0, 0)),
            pl.BlockSpec((1, D), lambda i: (0, 0)),
            pl.BlockSpec((1, D), lambda i: (0, 0)),
            pl.BlockSpec((1, D), lambda i: (0, 0)),
        ],
        out_specs=pl.BlockSpec((2, RN, DH), lambda i: (0, i, 0)),
        out_shape=jax.ShapeDtypeStruct((2, N, DH), jnp.float32),
    )(h, w0, w1, c, g, b)


def _sage_body(hh_ref, agg_ref, deg_ref, ws_ref, bs_ref, wn_ref, g_ref,
               be_ref, o_ref):
    eps = 1e-5
    deg = deg_ref[0, :, 0:1] + deg_ref[1, :, 0:1]
    inv = jnp.where(deg > 0, 1.0 / jnp.maximum(deg, 1.0), 0.0)
    x0 = hh_ref[0]
    x1 = hh_ref[1]
    m0 = agg_ref[0] * inv
    m1 = agg_ref[1] * inv
    halves = []
    for j in (0, 1):
        r = (_dot(x0, ws_ref[j * DH:(j + 1) * DH, 0:DH])
             + _dot(x1, ws_ref[j * DH:(j + 1) * DH, DH:D])
             + _dot(m0, wn_ref[j * DH:(j + 1) * DH, 0:DH])
             + _dot(m1, wn_ref[j * DH:(j + 1) * DH, DH:D])
             + bs_ref[0, j * DH:(j + 1) * DH])
        halves.append(jnp.maximum(r, 0.0))
    r0, r1 = halves
    mu = (jnp.sum(r0, axis=1, keepdims=True)
          + jnp.sum(r1, axis=1, keepdims=True)) / D
    c0 = r0 - mu
    c1 = r1 - mu
    var = (jnp.sum(c0 * c0, axis=1, keepdims=True)
           + jnp.sum(c1 * c1, axis=1, keepdims=True)) / D
    rs = lax.rsqrt(var + eps)
    o_ref[0] = c0 * rs * g_ref[0, 0:DH] + be_ref[0, 0:DH]
    o_ref[1] = c1 * rs * g_ref[0, DH:D] + be_ref[0, DH:D]


def _sage_layer(hh, agg, degp, ws, bs, wn, g, be):
    grid = (N // RN,)
    return pl.pallas_call(
        _sage_body,
        grid=grid,
        in_specs=[
            pl.BlockSpec((2, RN, DH), lambda i: (0, i, 0)),
            pl.BlockSpec((2, RN, DH), lambda i: (0, i, 0)),
            pl.BlockSpec((2, RN, DH), lambda i: (0, i, 0)),
            pl.BlockSpec((D, D), lambda i: (0, 0)),
            pl.BlockSpec((1, D), lambda i: (0, 0)),
            pl.BlockSpec((D, D), lambda i: (0, 0)),
            pl.BlockSpec((1, D), lambda i: (0, 0)),
            pl.BlockSpec((1, D), lambda i: (0, 0)),
        ],
        out_specs=pl.BlockSpec((2, RN, DH), lambda i: (0, i, 0)),
        out_shape=jax.ShapeDtypeStruct((2, N, DH), jnp.float32),
    )(hh, agg, degp, ws, bs, wn, g, be)


def _pred_tables_body(hh_ref, w1_ref, o_ref):
    x0 = hh_ref[0]
    x1 = hh_ref[1]
    for t in range(4):
        off = (t // 2) * D
        j = t % 2
        o_ref[t] = (_dot(x0, w1_ref[j * DH:(j + 1) * DH, off:off + DH])
                    + _dot(x1, w1_ref[j * DH:(j + 1) * DH, off + DH:off + D]))


def _pred_tables(hh, w1):
    grid = (N // RN,)
    return pl.pallas_call(
        _pred_tables_body,
        grid=grid,
        in_specs=[
            pl.BlockSpec((2, RN, DH), lambda i: (0, i, 0)),
            pl.BlockSpec((D, 2 * D), lambda i: (0, 0)),
        ],
        out_specs=pl.BlockSpec((4, RN, DH), lambda i: (0, i, 0)),
        out_shape=jax.ShapeDtypeStruct((4, N, DH), jnp.float32),
    )(hh, w1)


def _edge_head_body(x_ref, ef_ref, b1_ref, g_ref, be_ref, w2_ref, b2_ref,
                    o_ref):
    eps = 1e-5
    x0 = x_ref[0] + b1_ref[0, 0:DH]
    x1 = x_ref[1] + b1_ref[0, DH:D]
    mu = (jnp.sum(x0, axis=1, keepdims=True)
          + jnp.sum(x1, axis=1, keepdims=True)) / D
    c0 = x0 - mu
    c1 = x1 - mu
    var = (jnp.sum(c0 * c0, axis=1, keepdims=True)
           + jnp.sum(c1 * c1, axis=1, keepdims=True)) / D
    rs = lax.rsqrt(var + eps)
    n0 = jnp.maximum(c0 * rs * g_ref[0, 0:DH] + be_ref[0, 0:DH], 0.0)
    n1 = jnp.maximum(c1 * rs * g_ref[0, DH:D] + be_ref[0, DH:D], 0.0)
    o_ref[...] = (_dot(n0, w2_ref[:, 0:DH])
                  + _dot(n1, w2_ref[:, DH:D])
                  + _dot(ef_ref[...], w2_ref[:, D:D + 2])
                  + b2_ref[0, :])


def _edge_head(x, ef, b1, g, be, w2, b2):
    grid = (E // RE,)
    return pl.pallas_call(
        _edge_head_body,
        grid=grid,
        in_specs=[
            pl.BlockSpec((2, RE, DH), lambda i: (0, i, 0)),
            pl.BlockSpec((RE, 2), lambda i: (i, 0)),
            pl.BlockSpec((1, D), lambda i: (0, 0)),
            pl.BlockSpec((1, D), lambda i: (0, 0)),
            pl.BlockSpec((1, D), lambda i: (0, 0)),
            pl.BlockSpec((8, D + 2), lambda i: (0, 0)),
            pl.BlockSpec((1, 8), lambda i: (0, 0)),
        ],
        out_specs=pl.BlockSpec((RE, 8), lambda i: (i, 0)),
        out_shape=jax.ShapeDtypeStruct((E, 8), jnp.float32),
    )(x, ef, b1, g, be, w2, b2)


def _sc_mesh():
    return plsc.VectorSubcoreMesh(core_axis_name="c", subcore_axis_name="s",
                                  num_cores=2, num_subcores=NSUB)


def _sc_deg(dst, ones_rows, zeros128):
    @functools.partial(
        pl.kernel,
        mesh=_sc_mesh(),
        out_type=jax.ShapeDtypeStruct((2, NP, DH), jnp.float32),
        scratch_types=[
            pltpu.VMEM((CD,), jnp.int32),
            pltpu.VMEM((CD, DH), jnp.float32),
            pltpu.VMEM_SHARED((NP, DH), jnp.float32),
        ],
    )
    def k(dst_hbm, ones_hbm, z_hbm, out_hbm, idx_v, ones_v, acc):
        c = lax.axis_index("c")
        s = lax.axis_index("s")
        wid = s * 2 + c
        pltpu.sync_copy(ones_hbm, ones_v)
        pltpu.sync_copy(z_hbm, acc.at[pl.ds(s * NPS, NPS)])
        plsc.subcore_barrier()

        def body(t, carry):
            base = wid * EPS_D + t * CD
            pltpu.sync_copy(dst_hbm.at[pl.ds(base, CD)], idx_v)
            pltpu.sync_copy(ones_v, acc.at[idx_v], add=True)
            return carry

        lax.fori_loop(0, TD, body, 0)
        plsc.subcore_barrier()
        pltpu.sync_copy(acc.at[pl.ds(s * NPS, NPS)],
                        out_hbm.at[c, pl.ds(s * NPS, NPS)])

    return k(dst, ones_rows, zeros128)


def _sc_agg(table, src2, dst, w, zeros128):
    @functools.partial(
        pl.kernel,
        mesh=_sc_mesh(),
        out_type=jax.ShapeDtypeStruct((2 * NP, DH), jnp.float32),
        scratch_types=[
            pltpu.VMEM((CA,), jnp.int32),
            pltpu.VMEM((CA,), jnp.int32),
            pltpu.VMEM((CA,), jnp.float32),
            pltpu.VMEM((CA, DH), jnp.float32),
            pltpu.VMEM_SHARED((NP, DH), jnp.float32),
            pltpu.SemaphoreType.DMA,
        ],
    )
    def k(tab_hbm, src_hbm, dst_hbm, w_hbm, z_hbm, out_hbm,
          si_v, di_v, w_v, rows_v, acc, sem):
        c = lax.axis_index("c")
        s = lax.axis_index("s")
        pltpu.sync_copy(z_hbm, acc.at[pl.ds(s * NPS, NPS)])
        plsc.subcore_barrier()

        def chunk(t, carry):
            base = s * EPS_A + t * CA
            pltpu.sync_copy(src_hbm.at[pl.ds(c * E + base, CA)], si_v)
            pltpu.sync_copy(dst_hbm.at[pl.ds(base, CA)], di_v)
            pltpu.sync_copy(w_hbm.at[pl.ds(base, CA)], w_v)
            pltpu.async_copy(tab_hbm.at[si_v], rows_v, sem).wait()

            def grp(g, cc):
                wv = w_v[pl.ds(g * 16, 16)]
                for l in range(16):
                    e = g * 16 + l
                    bc = wv.at[jnp.full((16,), l, jnp.int32)].get(
                        mode="promise_in_bounds")
                    for f in range(DH // 16):
                        sl = pl.ds(f * 16, 16)
                        rows_v[e, sl] = rows_v[e, sl] * bc
                return cc

            lax.fori_loop(0, CA // 16, grp, 0)
            pltpu.sync_copy(rows_v, acc.at[di_v], add=True)
            return carry

        lax.fori_loop(0, TA, chunk, 0)
        plsc.subcore_barrier()
        pltpu.sync_copy(acc.at[pl.ds(s * NPS, NPS)],
                        out_hbm.at[pl.ds(c * NP + s * NPS, NPS)])

    return k(table, src2, dst, w, zeros128)


def _sc_edge_gather(table4, srcp, dstp):
    @functools.partial(
        pl.kernel,
        mesh=_sc_mesh(),
        out_type=jax.ShapeDtypeStruct((2, E, DH), jnp.float32),
        scratch_types=[
            pltpu.VMEM((CA,), jnp.int32),
            pltpu.VMEM((CA,), jnp.int32),
            pltpu.VMEM((CA, DH), jnp.float32),
            pltpu.VMEM((CA, DH), jnp.float32),
            pltpu.SemaphoreType.DMA,
            pltpu.SemaphoreType.DMA,
        ],
    )
    def k(tab_hbm, src_hbm, dst_hbm, out_hbm, si_v, di_v, ra_v, rb_v, sa, sb):
        c = lax.axis_index("c")
        s = lax.axis_index("s")

        def chunk(t, carry):
            base = s * EPS_A + t * CA
            pltpu.sync_copy(src_hbm.at[pl.ds(c * E + base, CA)], si_v)
            pltpu.sync_copy(dst_hbm.at[pl.ds(c * E + base, CA)], di_v)
            cpa = pltpu.async_copy(tab_hbm.at[si_v], ra_v, sa)
            cpb = pltpu.async_copy(tab_hbm.at[di_v], rb_v, sb)
            cpa.wait()
            cpb.wait()

            def erow(e, cc):
                for f in range(DH // 16):
                    sl = pl.ds(f * 16, 16)
                    ra_v[e, sl] = ra_v[e, sl] + rb_v[e, sl]
                return cc

            lax.fori_loop(0, CA, erow, 0)
            pltpu.sync_copy(ra_v, out_hbm.at[c, pl.ds(base, CA)])
            return carry

        lax.fori_loop(0, TA, chunk, 0)

    return k(table4, srcp, dstp)


def kernel(h, edge_weight, edge_feat, params, edge_index):
    p = params
    src = edge_index[0]
    dst = edge_index[1]

    c = jnp.concatenate([p['cp0'], p['cp1']])[None, :]
    g_in = jnp.concatenate([p['gp0'], p['gp1']])[None, :]
    b_in = jnp.concatenate([p['betap0'], p['betap1']])[None, :]
    hh = _input_proj(h, p['Wp0'], p['Wp1'], c, g_in, b_in)

    src2 = jnp.concatenate([src, src + N])
    dstp = jnp.concatenate([dst + 2 * N, dst + 3 * N])
    ones_rows = jnp.ones((CD, DH), jnp.float32)
    zeros128 = jnp.zeros((NPS, DH), jnp.float32)

    degp = _sc_deg(dst, ones_rows, zeros128)

    for l in range(3):
        agg2 = _sc_agg(hh.reshape(2 * N, DH), src2, dst, edge_weight,
                       zeros128).reshape(2, NP, DH)
        hh = _sage_layer(hh, agg2, degp, p[f'Wself{l}'], p[f'bself{l}'][None, :],
                         p[f'Wneigh{l}'], p[f'g{l}'][None, :],
                         p[f'beta{l}'][None, :])

    tab = _pred_tables(hh, p['W1'])
    x2 = _sc_edge_gather(tab.reshape(4 * N, DH), src2, dstp)

    w2p = jnp.zeros((8, D + 2), jnp.float32).at[:EDGE_CLASSES].set(p['W2'])
    b2p = jnp.zeros((1, 8), jnp.float32).at[0, :EDGE_CLASSES].set(p['b2'])
    out8 = _edge_head(x2, edge_feat, p['b1'][None, :], p['g_pred'][None, :],
                      p['beta_pred'][None, :], w2p, b2p)
    return out8[:, :EDGE_CLASSES]
